# baseline (speedup 1.0000x reference)
import functools
import math

import jax
import jax.numpy as jnp
from jax import lax
from jax.experimental import pallas as pl
from jax.experimental.pallas import tpu as pltpu
from jax.experimental.pallas import tpu_sc as plsc

_N = 9976
_E = 638464
_B = 86
_IN = 128
_OUT = 116
_DH = 58

_NP = 9984
_ROWS_PER_SUB = _NP // 16
_NW = 32
_PER_W = 20480
_EP = _NW * _PER_W
_CH = 64
_CHUNKS = _PER_W // _CH
_HALF = _CHUNKS // 2
_PAD_DST = _NP - 1
_INV_SQRT_DH = 1.0 / math.sqrt(float(_DH))

_GATHER_DN = lax.GatherDimensionNumbers(
    offset_dims=(), collapsed_slice_dims=(0,), start_index_map=(0,))


def _lane_shuffle(x, idx):
    return lax.gather(x, idx[:, None], _GATHER_DN, slice_sizes=(1,),
                      mode=lax.GatherScatterMode.PROMISE_IN_BOUNDS)


def _qkv_body(h_ref, w_ref, b_ref, q_ref, kv_ref):
    h = h_ref[...]
    q_ref[...] = jnp.dot(h, w_ref[:, 0:128],
                         preferred_element_type=jnp.float32) + b_ref[0:1, 0:128]
    kv_ref[:, 0:128] = jnp.dot(h, w_ref[:, 128:256],
                               preferred_element_type=jnp.float32) + b_ref[0:1, 128:256]
    kv_ref[:, 128:256] = jnp.dot(h, w_ref[:, 256:384],
                                 preferred_element_type=jnp.float32) + b_ref[0:1, 256:384]


_qkv_call = pl.pallas_call(
    _qkv_body,
    grid=(_NP // 128,),
    in_specs=[
        pl.BlockSpec((128, 128), lambda i: (i, 0)),
        pl.BlockSpec((128, 384), lambda i: (0, 0)),
        pl.BlockSpec((8, 384), lambda i: (0, 0)),
    ],
    out_specs=[
        pl.BlockSpec((128, 128), lambda i: (i, 0)),
        pl.BlockSpec((128, 256), lambda i: (i, 0)),
    ],
    out_shape=[
        jax.ShapeDtypeStruct((_NP, 128), jnp.float32),
        jax.ShapeDtypeStruct((_NP, 256), jnp.float32),
    ],
)


@functools.cache
def _get_edge_kernel():
    return functools.partial(
        pl.kernel,
        mesh=plsc.VectorSubcoreMesh(core_axis_name="c", subcore_axis_name="s"),
        out_type=[jax.ShapeDtypeStruct((_NP, 128), jnp.float32)] * 2,
        scratch_types=[
            pltpu.VMEM((_CH,), jnp.int32),
            pltpu.VMEM((_CH,), jnp.int32),
            pltpu.VMEM((_CH,), jnp.int32),
            pltpu.VMEM((_CH,), jnp.int32),
            pltpu.VMEM((_CH,), jnp.int32),
            pltpu.VMEM((_CH,), jnp.int32),
            pltpu.VMEM((_CH, 128), jnp.float32),
            pltpu.VMEM((_CH, 256), jnp.float32),
            pltpu.VMEM((_CH, 128), jnp.float32),
            pltpu.VMEM((_CH, 256), jnp.float32),
            pltpu.VMEM_SHARED((_NP, 128), jnp.float32),
            pltpu.SemaphoreType.DMA,
            pltpu.SemaphoreType.DMA,
            pltpu.SemaphoreType.DMA,
            pltpu.SemaphoreType.DMA,
            pltpu.SemaphoreType.DMA,
            pltpu.SemaphoreType.DMA,
        ],
    )(_edge_body)


def _compute_chunk(qb_, kvb_, lane):
    half = lane < 8
    swap8 = lane ^ 8

    def edge_body(eb, carry2):
        for j in range(8):
            e = eb * 8 + j
            t0 = (qb_[e, pl.ds(0, 16)] * kvb_[e, pl.ds(0, 16)]
                  + qb_[e, pl.ds(16, 16)] * kvb_[e, pl.ds(16, 16)]
                  + qb_[e, pl.ds(32, 16)] * kvb_[e, pl.ds(32, 16)]
                  + qb_[e, pl.ds(48, 16)] * kvb_[e, pl.ds(48, 16)])
            t1 = (qb_[e, pl.ds(64, 16)] * kvb_[e, pl.ds(64, 16)]
                  + qb_[e, pl.ds(80, 16)] * kvb_[e, pl.ds(80, 16)]
                  + qb_[e, pl.ds(96, 16)] * kvb_[e, pl.ds(96, 16)]
                  + qb_[e, pl.ds(112, 16)] * kvb_[e, pl.ds(112, 16)])
            t0 = t0 + _lane_shuffle(t0, swap8)
            t1 = t1 + _lane_shuffle(t1, swap8)
            w = jnp.where(half, t0, _lane_shuffle(t1, swap8))
            for dlt in (1, 2, 4):
                w = w + _lane_shuffle(w, lane ^ dlt)
            exw = jnp.exp(w * _INV_SQRT_DH)
            ex0 = _lane_shuffle(exw, lane & 0)
            ex1 = _lane_shuffle(exw, (lane & 0) + 8)
            qb_[e, pl.ds(0, 16)] = ex0 * kvb_[e, pl.ds(128, 16)]
            qb_[e, pl.ds(16, 16)] = ex0 * kvb_[e, pl.ds(144, 16)]
            qb_[e, pl.ds(32, 16)] = ex0 * kvb_[e, pl.ds(160, 16)]
            qb_[e, pl.ds(48, 16)] = jnp.where(
                lane == 10, ex0, ex0 * kvb_[e, pl.ds(176, 16)])
            qb_[e, pl.ds(64, 16)] = ex1 * kvb_[e, pl.ds(192, 16)]
            qb_[e, pl.ds(80, 16)] = ex1 * kvb_[e, pl.ds(208, 16)]
            qb_[e, pl.ds(96, 16)] = ex1 * kvb_[e, pl.ds(224, 16)]
            qb_[e, pl.ds(112, 16)] = jnp.where(
                lane == 10, ex1, ex1 * kvb_[e, pl.ds(240, 16)])
        return carry2

    lax.fori_loop(0, _CH // 8, edge_body, 0)


def _edge_body(q_hbm, kv_hbm, src_hbm, dst_hbm, zero_hbm,
               out0, out1, sidx0, didx0, sidx1, didx1, sdid0, sdid1,
               qb0, kvb0, qb1, kvb1,
               acc, sem_g0, sem_g1, sem_i0, sem_i1, sem_s0, sem_s1):
    c = lax.axis_index("c")
    s = lax.axis_index("s")

    pltpu.sync_copy(zero_hbm.at[pl.ds(s * _ROWS_PER_SUB, _ROWS_PER_SUB)],
                    acc.at[pl.ds(s * _ROWS_PER_SUB, _ROWS_PER_SUB)])
    plsc.subcore_barrier()

    base = (c * 16 + s) * _PER_W
    lane = lax.iota(jnp.int32, 16)

    def _snapshot(dst_ref, src_ref):
        for t in range(_CH // 16):
            dst_ref[pl.ds(t * 16, 16)] = src_ref[pl.ds(t * 16, 16)]

    pltpu.sync_copy(src_hbm.at[pl.ds(base, _CH)], sidx0)
    pltpu.sync_copy(dst_hbm.at[pl.ds(base, _CH)], didx0)
    pltpu.async_copy(q_hbm.at[didx0], qb0, sem_g0)
    pltpu.async_copy(kv_hbm.at[sidx0], kvb0, sem_g0)
    pltpu.async_copy(src_hbm.at[pl.ds(base + _CH, _CH)], sidx1, sem_i1)
    pltpu.async_copy(dst_hbm.at[pl.ds(base + _CH, _CH)], didx1, sem_i1)

    def half_body(i, carry):
        g0 = 2 * i
        pltpu.make_async_copy(
            src_hbm.at[pl.ds(base + (g0 + 1) * _CH, _CH)], sidx1, sem_i1).wait()
        pltpu.make_async_copy(
            dst_hbm.at[pl.ds(base + (g0 + 1) * _CH, _CH)], didx1, sem_i1).wait()

        @pl.when(i > 0)
        def _():
            pltpu.make_async_copy(qb1, acc.at[sdid1], sem_s1).wait()

        pltpu.async_copy(q_hbm.at[didx1], qb1, sem_g1)
        pltpu.async_copy(kv_hbm.at[sidx1], kvb1, sem_g1)

        pltpu.make_async_copy(q_hbm.at[didx0], qb0, sem_g0).wait()
        pltpu.make_async_copy(kv_hbm.at[sidx0], kvb0, sem_g0).wait()
        _snapshot(sdid0, didx0)

        @pl.when(i + 1 < _HALF)
        def _():
            off2 = base + (g0 + 2) * _CH
            pltpu.async_copy(src_hbm.at[pl.ds(off2, _CH)], sidx0, sem_i0)
            pltpu.async_copy(dst_hbm.at[pl.ds(off2, _CH)], didx0, sem_i0)

        _compute_chunk(qb0, kvb0, lane)
        pltpu.async_copy(qb0, acc.at[sdid0], sem_s0, add=True)

        @pl.when(i + 1 < _HALF)
        def _():
            off2 = base + (g0 + 2) * _CH
            pltpu.make_async_copy(
                src_hbm.at[pl.ds(off2, _CH)], sidx0, sem_i0).wait()
            pltpu.make_async_copy(
                dst_hbm.at[pl.ds(off2, _CH)], didx0, sem_i0).wait()

        pltpu.make_async_copy(qb0, acc.at[sdid0], sem_s0).wait()

        @pl.when(i + 1 < _HALF)
        def _():
            pltpu.async_copy(q_hbm.at[didx0], qb0, sem_g0)
            pltpu.async_copy(kv_hbm.at[sidx0], kvb0, sem_g0)

        pltpu.make_async_copy(q_hbm.at[didx1], qb1, sem_g1).wait()
        pltpu.make_async_copy(kv_hbm.at[sidx1], kvb1, sem_g1).wait()
        _snapshot(sdid1, didx1)

        @pl.when(i + 1 < _HALF)
        def _():
            off3 = base + (g0 + 3) * _CH
            pltpu.async_copy(src_hbm.at[pl.ds(off3, _CH)], sidx1, sem_i1)
            pltpu.async_copy(dst_hbm.at[pl.ds(off3, _CH)], didx1, sem_i1)

        _compute_chunk(qb1, kvb1, lane)
        pltpu.async_copy(qb1, acc.at[sdid1], sem_s1, add=True)
        return carry

    lax.fori_loop(0, _HALF, half_body, 0)
    pltpu.make_async_copy(qb1, acc.at[sdid1], sem_s1).wait()
    plsc.subcore_barrier()

    @pl.when(c == 0)
    def _():
        pltpu.sync_copy(acc.at[pl.ds(s * _ROWS_PER_SUB, _ROWS_PER_SUB)],
                        out0.at[pl.ds(s * _ROWS_PER_SUB, _ROWS_PER_SUB)])

    @pl.when(c == 1)
    def _():
        pltpu.sync_copy(acc.at[pl.ds(s * _ROWS_PER_SUB, _ROWS_PER_SUB)],
                        out1.at[pl.ds(s * _ROWS_PER_SUB, _ROWS_PER_SUB)])


def _proj_body(a0_ref, a1_ref, w_ref, b_ref, o_ref):
    sacc = a0_ref[...] + a1_ref[...]
    col = lax.broadcasted_iota(jnp.int32, (128, 128), 1)
    den = jnp.where(col < 64, sacc[:, 58:59], sacc[:, 122:123]) + 1e-16
    hat = sacc / den
    o_ref[...] = jnp.dot(hat, w_ref[...],
                         preferred_element_type=jnp.float32) + b_ref[0:1, :]


_proj_call = pl.pallas_call(
    _proj_body,
    grid=(_NP // 128,),
    in_specs=[
        pl.BlockSpec((128, 128), lambda i: (i, 0)),
        pl.BlockSpec((128, 128), lambda i: (i, 0)),
        pl.BlockSpec((128, 128), lambda i: (0, 0)),
        pl.BlockSpec((8, 128), lambda i: (0, 0)),
    ],
    out_specs=pl.BlockSpec((128, 128), lambda i: (i, 0)),
    out_shape=jax.ShapeDtypeStruct((_NP, 128), jnp.float32),
)


def _att_body(x_ref, a_ref, o_ref):
    x = x_ref[...]
    e3 = jnp.sum(x, axis=2) * (1.0 / float(_OUT))
    row = lax.broadcasted_iota(jnp.int32, (_B, 120), 1)
    e3 = jnp.where(row < _OUT, e3, -jnp.inf)
    m = jnp.max(e3, axis=1, keepdims=True)
    ex = jnp.exp(e3 - m)
    den = jnp.sum(ex, axis=1, keepdims=True) + 1e-16
    alpha = ex / den
    a_ref[...] = alpha
    o_ref[...] = alpha[:, :, None] * x


_att_call = pl.pallas_call(
    _att_body,
    in_specs=[pl.BlockSpec((_B, 120, 128), lambda: (0, 0, 0))],
    out_specs=[
        pl.BlockSpec((_B, 120), lambda: (0, 0)),
        pl.BlockSpec((_B, 120, 128), lambda: (0, 0, 0)),
    ],
    out_shape=[
        jax.ShapeDtypeStruct((_B, 120), jnp.float32),
        jax.ShapeDtypeStruct((_B, 120, 128), jnp.float32),
    ],
)


def _headpad_w(W):
    Wp = jnp.zeros((W.shape[0], 128), jnp.float32)
    Wp = Wp.at[:, 0:_DH].set(W[:, 0:_DH])
    Wp = Wp.at[:, 64:64 + _DH].set(W[:, _DH:_OUT])
    return Wp


def _headpad_b(b):
    bp = jnp.zeros((128,), jnp.float32)
    bp = bp.at[0:_DH].set(b[0:_DH])
    bp = bp.at[64:64 + _DH].set(b[_DH:_OUT])
    return bp


def kernel(h_flat, edge_index, batch_index, WQ, bQ, WK, bK, WV, bV, WO, bO):
    f32 = jnp.float32
    hp = jnp.zeros((_NP, _IN), f32).at[:_N].set(h_flat.astype(f32))

    Wcat = jnp.concatenate([_headpad_w(WQ), _headpad_w(WK), _headpad_w(WV)],
                           axis=1)
    bcat = jnp.zeros((8, 384), f32).at[0].set(
        jnp.concatenate([_headpad_b(bQ), _headpad_b(bK), _headpad_b(bV)]))

    qp, kvp = _qkv_call(hp, Wcat, bcat)

    src_p = jnp.concatenate(
        [edge_index[0], jnp.zeros((_EP - _E,), jnp.int32)])
    dst_p = jnp.concatenate(
        [edge_index[1], jnp.full((_EP - _E,), _PAD_DST, jnp.int32)])
    zeros_tab = jnp.zeros((_NP, 128), f32)

    acc0, acc1 = _get_edge_kernel()(qp, kvp, src_p, dst_p, zeros_tab)

    WOp = (jnp.zeros((128, 128), f32)
           .at[0:_DH, 0:_OUT].set(WO[0:_DH])
           .at[64:64 + _DH, 0:_OUT].set(WO[_DH:_OUT]))
    bOp = jnp.zeros((8, 128), f32).at[0, 0:_OUT].set(bO)

    hpj = _proj_call(acc0, acc1, WOp, bOp)

    x4 = jnp.pad(hpj[:_N].reshape(_B, _OUT, 128), ((0, 0), (0, 4), (0, 0)))
    am, hw = _att_call(x4)

    alpha_map = am[:, :_OUT]
    h_weighted = hw[:, :_OUT, :_OUT].reshape(_N, _OUT)
    return (alpha_map, h_weighted)


# speedup vs baseline: 1.0236x; 1.0236x over previous
import functools
import math

import jax
import jax.numpy as jnp
from jax import lax
from jax.experimental import pallas as pl
from jax.experimental.pallas import tpu as pltpu
from jax.experimental.pallas import tpu_sc as plsc

_N = 9976
_E = 638464
_B = 86
_IN = 128
_OUT = 116
_DH = 58

_NP = 9984
_ROWS_PER_SUB = _NP // 16
_NW = 32
_PER_W = 20480
_EP = _NW * _PER_W
_CH = 64
_CHUNKS = _PER_W // _CH
_HALF = _CHUNKS // 2
_PAD_DST = _NP - 1
_INV_SQRT_DH = 1.0 / math.sqrt(float(_DH))

_GATHER_DN = lax.GatherDimensionNumbers(
    offset_dims=(), collapsed_slice_dims=(0,), start_index_map=(0,))


def _lane_shuffle(x, idx):
    return lax.gather(x, idx[:, None], _GATHER_DN, slice_sizes=(1,),
                      mode=lax.GatherScatterMode.PROMISE_IN_BOUNDS)


def _qkv_body(h_ref, w_ref, b_ref, q_ref, kv_ref):
    h = h_ref[...]
    q_ref[...] = jnp.dot(h, w_ref[:, 0:128],
                         preferred_element_type=jnp.float32) + b_ref[0:1, 0:128]
    kv_ref[:, 0:128] = jnp.dot(h, w_ref[:, 128:256],
                               preferred_element_type=jnp.float32) + b_ref[0:1, 128:256]
    kv_ref[:, 128:256] = jnp.dot(h, w_ref[:, 256:384],
                                 preferred_element_type=jnp.float32) + b_ref[0:1, 256:384]


_qkv_call = pl.pallas_call(
    _qkv_body,
    grid=(_NP // 128,),
    in_specs=[
        pl.BlockSpec((128, 128), lambda i: (i, 0)),
        pl.BlockSpec((128, 384), lambda i: (0, 0)),
        pl.BlockSpec((8, 384), lambda i: (0, 0)),
    ],
    out_specs=[
        pl.BlockSpec((128, 128), lambda i: (i, 0)),
        pl.BlockSpec((128, 256), lambda i: (i, 0)),
    ],
    out_shape=[
        jax.ShapeDtypeStruct((_NP, 128), jnp.float32),
        jax.ShapeDtypeStruct((_NP, 256), jnp.float32),
    ],
)


@functools.cache
def _get_edge_kernel():
    return functools.partial(
        pl.kernel,
        mesh=plsc.VectorSubcoreMesh(core_axis_name="c", subcore_axis_name="s"),
        out_type=[jax.ShapeDtypeStruct((_NP, 128), jnp.float32)] * 2,
        scratch_types=[
            pltpu.VMEM((_CH,), jnp.int32),
            pltpu.VMEM((_CH,), jnp.int32),
            pltpu.VMEM((_CH,), jnp.int32),
            pltpu.VMEM((_CH,), jnp.int32),
            pltpu.VMEM((_CH,), jnp.int32),
            pltpu.VMEM((_CH,), jnp.int32),
            pltpu.VMEM((_CH, 128), jnp.float32),
            pltpu.VMEM((_CH, 256), jnp.float32),
            pltpu.VMEM((_CH, 128), jnp.float32),
            pltpu.VMEM((_CH, 256), jnp.float32),
            pltpu.VMEM_SHARED((_NP, 128), jnp.float32),
            pltpu.SemaphoreType.DMA,
            pltpu.SemaphoreType.DMA,
            pltpu.SemaphoreType.DMA,
            pltpu.SemaphoreType.DMA,
            pltpu.SemaphoreType.DMA,
            pltpu.SemaphoreType.DMA,
        ],
    )(_edge_body)


def _compute_chunk(qb_, kvb_, lane):
    def edge_body(eb, carry2):
        for j in range(4):
            e = eb * 4 + j
            t0 = (qb_[e, pl.ds(0, 16)] * kvb_[e, pl.ds(0, 16)]
                  + qb_[e, pl.ds(16, 16)] * kvb_[e, pl.ds(16, 16)]
                  + qb_[e, pl.ds(32, 16)] * kvb_[e, pl.ds(32, 16)]
                  + qb_[e, pl.ds(48, 16)] * kvb_[e, pl.ds(48, 16)])
            t1 = (qb_[e, pl.ds(64, 16)] * kvb_[e, pl.ds(64, 16)]
                  + qb_[e, pl.ds(80, 16)] * kvb_[e, pl.ds(80, 16)]
                  + qb_[e, pl.ds(96, 16)] * kvb_[e, pl.ds(96, 16)]
                  + qb_[e, pl.ds(112, 16)] * kvb_[e, pl.ds(112, 16)])
            for dlt in (1, 2, 4, 8):
                prm = lane ^ dlt
                t0 = t0 + _lane_shuffle(t0, prm)
                t1 = t1 + _lane_shuffle(t1, prm)
            ex0 = jnp.exp(t0 * _INV_SQRT_DH)
            ex1 = jnp.exp(t1 * _INV_SQRT_DH)
            qb_[e, pl.ds(0, 16)] = ex0 * kvb_[e, pl.ds(128, 16)]
            qb_[e, pl.ds(16, 16)] = ex0 * kvb_[e, pl.ds(144, 16)]
            qb_[e, pl.ds(32, 16)] = ex0 * kvb_[e, pl.ds(160, 16)]
            qb_[e, pl.ds(48, 16)] = jnp.where(
                lane == 10, ex0, ex0 * kvb_[e, pl.ds(176, 16)])
            qb_[e, pl.ds(64, 16)] = ex1 * kvb_[e, pl.ds(192, 16)]
            qb_[e, pl.ds(80, 16)] = ex1 * kvb_[e, pl.ds(208, 16)]
            qb_[e, pl.ds(96, 16)] = ex1 * kvb_[e, pl.ds(224, 16)]
            qb_[e, pl.ds(112, 16)] = jnp.where(
                lane == 10, ex1, ex1 * kvb_[e, pl.ds(240, 16)])
        return carry2

    lax.fori_loop(0, _CH // 4, edge_body, 0)


def _edge_body(q_hbm, kv_hbm, src_hbm, dst_hbm, zero_hbm,
               out0, out1, sidx0, didx0, sidx1, didx1, sdid0, sdid1,
               qb0, kvb0, qb1, kvb1,
               acc, sem_g0, sem_g1, sem_i0, sem_i1, sem_s0, sem_s1):
    c = lax.axis_index("c")
    s = lax.axis_index("s")

    pltpu.sync_copy(zero_hbm.at[pl.ds(s * _ROWS_PER_SUB, _ROWS_PER_SUB)],
                    acc.at[pl.ds(s * _ROWS_PER_SUB, _ROWS_PER_SUB)])
    plsc.subcore_barrier()

    base = (c * 16 + s) * _PER_W
    lane = lax.iota(jnp.int32, 16)

    def _snapshot(dst_ref, src_ref):
        for t in range(_CH // 16):
            dst_ref[pl.ds(t * 16, 16)] = src_ref[pl.ds(t * 16, 16)]

    pltpu.sync_copy(src_hbm.at[pl.ds(base, _CH)], sidx0)
    pltpu.sync_copy(dst_hbm.at[pl.ds(base, _CH)], didx0)
    pltpu.async_copy(q_hbm.at[didx0], qb0, sem_g0)
    pltpu.async_copy(kv_hbm.at[sidx0], kvb0, sem_g0)
    pltpu.async_copy(src_hbm.at[pl.ds(base + _CH, _CH)], sidx1, sem_i1)
    pltpu.async_copy(dst_hbm.at[pl.ds(base + _CH, _CH)], didx1, sem_i1)

    def half_body(i, carry):
        g0 = 2 * i
        pltpu.make_async_copy(
            src_hbm.at[pl.ds(base + (g0 + 1) * _CH, _CH)], sidx1, sem_i1).wait()
        pltpu.make_async_copy(
            dst_hbm.at[pl.ds(base + (g0 + 1) * _CH, _CH)], didx1, sem_i1).wait()

        @pl.when(i > 0)
        def _():
            pltpu.make_async_copy(qb1, acc.at[sdid1], sem_s1).wait()

        pltpu.async_copy(q_hbm.at[didx1], qb1, sem_g1)
        pltpu.async_copy(kv_hbm.at[sidx1], kvb1, sem_g1)

        pltpu.make_async_copy(q_hbm.at[didx0], qb0, sem_g0).wait()
        pltpu.make_async_copy(kv_hbm.at[sidx0], kvb0, sem_g0).wait()
        _snapshot(sdid0, didx0)

        @pl.when(i + 1 < _HALF)
        def _():
            off2 = base + (g0 + 2) * _CH
            pltpu.async_copy(src_hbm.at[pl.ds(off2, _CH)], sidx0, sem_i0)
            pltpu.async_copy(dst_hbm.at[pl.ds(off2, _CH)], didx0, sem_i0)

        _compute_chunk(qb0, kvb0, lane)
        pltpu.async_copy(qb0, acc.at[sdid0], sem_s0, add=True)

        @pl.when(i + 1 < _HALF)
        def _():
            off2 = base + (g0 + 2) * _CH
            pltpu.make_async_copy(
                src_hbm.at[pl.ds(off2, _CH)], sidx0, sem_i0).wait()
            pltpu.make_async_copy(
                dst_hbm.at[pl.ds(off2, _CH)], didx0, sem_i0).wait()

        pltpu.make_async_copy(qb0, acc.at[sdid0], sem_s0).wait()

        @pl.when(i + 1 < _HALF)
        def _():
            pltpu.async_copy(q_hbm.at[didx0], qb0, sem_g0)
            pltpu.async_copy(kv_hbm.at[sidx0], kvb0, sem_g0)

        pltpu.make_async_copy(q_hbm.at[didx1], qb1, sem_g1).wait()
        pltpu.make_async_copy(kv_hbm.at[sidx1], kvb1, sem_g1).wait()
        _snapshot(sdid1, didx1)

        @pl.when(i + 1 < _HALF)
        def _():
            off3 = base + (g0 + 3) * _CH
            pltpu.async_copy(src_hbm.at[pl.ds(off3, _CH)], sidx1, sem_i1)
            pltpu.async_copy(dst_hbm.at[pl.ds(off3, _CH)], didx1, sem_i1)

        _compute_chunk(qb1, kvb1, lane)
        pltpu.async_copy(qb1, acc.at[sdid1], sem_s1, add=True)
        return carry

    lax.fori_loop(0, _HALF, half_body, 0)
    pltpu.make_async_copy(qb1, acc.at[sdid1], sem_s1).wait()
    plsc.subcore_barrier()

    @pl.when(c == 0)
    def _():
        pltpu.sync_copy(acc.at[pl.ds(s * _ROWS_PER_SUB, _ROWS_PER_SUB)],
                        out0.at[pl.ds(s * _ROWS_PER_SUB, _ROWS_PER_SUB)])

    @pl.when(c == 1)
    def _():
        pltpu.sync_copy(acc.at[pl.ds(s * _ROWS_PER_SUB, _ROWS_PER_SUB)],
                        out1.at[pl.ds(s * _ROWS_PER_SUB, _ROWS_PER_SUB)])


def _proj_body(a0_ref, a1_ref, w_ref, b_ref, o_ref):
    sacc = a0_ref[...] + a1_ref[...]
    col = lax.broadcasted_iota(jnp.int32, (128, 128), 1)
    den = jnp.where(col < 64, sacc[:, 58:59], sacc[:, 122:123]) + 1e-16
    hat = sacc / den
    o_ref[...] = jnp.dot(hat, w_ref[...],
                         preferred_element_type=jnp.float32) + b_ref[0:1, :]


_proj_call = pl.pallas_call(
    _proj_body,
    grid=(_NP // 128,),
    in_specs=[
        pl.BlockSpec((128, 128), lambda i: (i, 0)),
        pl.BlockSpec((128, 128), lambda i: (i, 0)),
        pl.BlockSpec((128, 128), lambda i: (0, 0)),
        pl.BlockSpec((8, 128), lambda i: (0, 0)),
    ],
    out_specs=pl.BlockSpec((128, 128), lambda i: (i, 0)),
    out_shape=jax.ShapeDtypeStruct((_NP, 128), jnp.float32),
)


def _att_body(x_ref, a_ref, o_ref):
    x = x_ref[...]
    e3 = jnp.sum(x, axis=2) * (1.0 / float(_OUT))
    row = lax.broadcasted_iota(jnp.int32, (_B, 120), 1)
    e3 = jnp.where(row < _OUT, e3, -jnp.inf)
    m = jnp.max(e3, axis=1, keepdims=True)
    ex = jnp.exp(e3 - m)
    den = jnp.sum(ex, axis=1, keepdims=True) + 1e-16
    alpha = ex / den
    a_ref[...] = alpha
    o_ref[...] = alpha[:, :, None] * x


_att_call = pl.pallas_call(
    _att_body,
    in_specs=[pl.BlockSpec((_B, 120, 128), lambda: (0, 0, 0))],
    out_specs=[
        pl.BlockSpec((_B, 120), lambda: (0, 0)),
        pl.BlockSpec((_B, 120, 128), lambda: (0, 0, 0)),
    ],
    out_shape=[
        jax.ShapeDtypeStruct((_B, 120), jnp.float32),
        jax.ShapeDtypeStruct((_B, 120, 128), jnp.float32),
    ],
)


def _headpad_w(W):
    Wp = jnp.zeros((W.shape[0], 128), jnp.float32)
    Wp = Wp.at[:, 0:_DH].set(W[:, 0:_DH])
    Wp = Wp.at[:, 64:64 + _DH].set(W[:, _DH:_OUT])
    return Wp


def _headpad_b(b):
    bp = jnp.zeros((128,), jnp.float32)
    bp = bp.at[0:_DH].set(b[0:_DH])
    bp = bp.at[64:64 + _DH].set(b[_DH:_OUT])
    return bp


def kernel(h_flat, edge_index, batch_index, WQ, bQ, WK, bK, WV, bV, WO, bO):
    f32 = jnp.float32
    hp = jnp.zeros((_NP, _IN), f32).at[:_N].set(h_flat.astype(f32))

    Wcat = jnp.concatenate([_headpad_w(WQ), _headpad_w(WK), _headpad_w(WV)],
                           axis=1)
    bcat = jnp.zeros((8, 384), f32).at[0].set(
        jnp.concatenate([_headpad_b(bQ), _headpad_b(bK), _headpad_b(bV)]))

    qp, kvp = _qkv_call(hp, Wcat, bcat)

    src_p = jnp.concatenate(
        [edge_index[0], jnp.zeros((_EP - _E,), jnp.int32)])
    dst_p = jnp.concatenate(
        [edge_index[1], jnp.full((_EP - _E,), _PAD_DST, jnp.int32)])
    zeros_tab = jnp.zeros((_NP, 128), f32)

    acc0, acc1 = _get_edge_kernel()(qp, kvp, src_p, dst_p, zeros_tab)

    WOp = (jnp.zeros((128, 128), f32)
           .at[0:_DH, 0:_OUT].set(WO[0:_DH])
           .at[64:64 + _DH, 0:_OUT].set(WO[_DH:_OUT]))
    bOp = jnp.zeros((8, 128), f32).at[0, 0:_OUT].set(bO)

    hpj = _proj_call(acc0, acc1, WOp, bOp)

    x4 = jnp.pad(hpj[:_N].reshape(_B, _OUT, 128), ((0, 0), (0, 4), (0, 0)))
    am, hw = _att_call(x4)

    alpha_map = am[:, :_OUT]
    h_weighted = hw[:, :_OUT, :_OUT].reshape(_N, _OUT)
    return (alpha_map, h_weighted)
